# baseline (speedup 1.0000x reference)
import functools

import jax
import jax.numpy as jnp
from jax import lax
from jax.experimental import pallas as pl
from jax.experimental.pallas import tpu as pltpu
from jax.experimental.pallas import tpu_sc as plsc

N_FACTORS = 32
BATCH = 16384
NC = 2
NS = 16
NW = NC * NS
BPW = BATCH // NW
CHUNK = 128
NCH = BPW // CHUNK
LANES = 16


def _mf_body(user_r, item_r, uf_r, if_r, out_r,
             uidx, iidx, urows, irows, outv, sem):
    wid = lax.axis_index("s") * NC + lax.axis_index("c")

    pltpu.sync_copy(user_r.at[wid], uidx)
    pltpu.sync_copy(item_r.at[wid], iidx)

    copies = []
    for j in range(NCH):
        dst_u = urows.at[pl.ds(j * CHUNK, CHUNK)]
        dst_i = irows.at[pl.ds(j * CHUNK, CHUNK)]
        copies.append(pltpu.async_copy(uf_r.at[uidx.at[j]], dst_u, sem))
        copies.append(pltpu.async_copy(if_r.at[iidx.at[j]], dst_i, sem))
    for c in copies:
        c.wait()

    lane = lax.iota(jnp.int32, LANES)

    def group(g, carry):
        acc = jnp.zeros((LANES,), jnp.float32)
        for r in range(LANES):
            row = g * LANES + r
            s0 = urows[row, pl.ds(0, LANES)] * irows[row, pl.ds(0, LANES)]
            s1 = urows[row, pl.ds(LANES, LANES)] * irows[row, pl.ds(LANES, LANES)]
            tot = jnp.sum(s0 + s1)
            acc = jnp.where(lane == r, tot, acc)
        outv[pl.ds(g * LANES, LANES)] = acc
        return carry

    lax.fori_loop(0, BPW // LANES, group, 0)

    pltpu.sync_copy(outv, out_r.at[wid])


_mf = functools.partial(
    pl.kernel,
    mesh=plsc.VectorSubcoreMesh(core_axis_name="c", subcore_axis_name="s"),
    out_type=jax.ShapeDtypeStruct((NW, BPW), jnp.float32),
    scratch_types=[
        pltpu.VMEM((NCH, CHUNK), jnp.int32),
        pltpu.VMEM((NCH, CHUNK), jnp.int32),
        pltpu.VMEM((BPW, N_FACTORS), jnp.float32),
        pltpu.VMEM((BPW, N_FACTORS), jnp.float32),
        pltpu.VMEM((BPW,), jnp.float32),
        pltpu.SemaphoreType.DMA,
    ],
    compiler_params=pltpu.CompilerParams(
        needs_layout_passes=False,
        use_tc_tiling_on_sc=False,
    ),
)(_mf_body)


def kernel(user, item, user_factors, item_factors):
    u = user.astype(jnp.int32).reshape(NW, NCH, CHUNK)
    i = item.astype(jnp.int32).reshape(NW, NCH, CHUNK)
    out = _mf(u, i, user_factors, item_factors)
    return out.reshape(BATCH)
